# Initial kernel scaffold; baseline (speedup 1.0000x reference)
import jax
import jax.numpy as jnp
from jax.experimental import pallas as pl


def kernel(nf, edge_index, W_init, b_init, W_lin0, W_attn0, W_lin1, W_attn1):
    raise NotImplementedError("write your pallas kernel here")


# speedup vs baseline: 13.4763x; 13.4763x over previous
import functools

import jax
import jax.numpy as jnp
from jax import lax
from jax.experimental import pallas as pl
from jax.experimental.pallas import tpu as pltpu
from jax.experimental.pallas import tpu_sc as plsc

N = 10000
E = 320000
H = 128
DT = 144
NPAD = 10240
TRASH = N
NW = 32
CHUNK = 128
R = 79
STRIPE = NPAD // 16
ZROWS = 80
BN = 400

@functools.cache
def _build_sc_edge_agg():
    mesh = plsc.VectorSubcoreMesh(core_axis_name="c", subcore_axis_name="s")

    @functools.partial(
        pl.kernel,
        out_type=jax.ShapeDtypeStruct((2 * NPAD, DT), jnp.float32),
        mesh=mesh,
        scratch_types=[
            pltpu.VMEM_SHARED((NPAD, DT), jnp.float32),
            pltpu.VMEM((R, CHUNK), jnp.int32),
            pltpu.VMEM((R, CHUNK), jnp.int32),
            pltpu.VMEM((CHUNK, DT), jnp.float32),
            pltpu.SemaphoreType.DMA,
        ],
        compiler_params=pltpu.CompilerParams(use_tc_tiling_on_sc=False),
    )
    def sc_body(t_hbm, srcp_hbm, dstp_hbm, z_hbm, out_hbm,
                acc, src_v, dst_v, rows_v, sem):
        c = lax.axis_index("c")
        s = lax.axis_index("s")
        wid = s * 2 + c
        base = s * STRIPE
        for j in range(STRIPE // ZROWS):
            pltpu.sync_copy(z_hbm, acc.at[pl.ds(base + j * ZROWS, ZROWS)])
        pltpu.sync_copy(srcp_hbm.at[wid], src_v)
        pltpu.sync_copy(dstp_hbm.at[wid], dst_v)
        plsc.subcore_barrier()

        def body(r, carry):
            pltpu.async_copy(t_hbm.at[src_v.at[r]], rows_v, sem).wait()
            pltpu.sync_copy(rows_v, acc.at[dst_v.at[r]], add=True)
            return carry

        lax.fori_loop(0, R, body, 0)
        plsc.subcore_barrier()
        pltpu.sync_copy(acc.at[pl.ds(base, STRIPE)],
                        out_hbm.at[pl.ds(c * NPAD + base, STRIPE)])

    return sc_body


def _sc_edge_agg(T, srcp, dstp, zblk):
    return _build_sc_edge_agg()(T, srcp, dstp, zblk)


def _tc_init(nf, W, b):
    def body(nf_ref, w_ref, b_ref, o_ref):
        o_ref[...] = nf_ref[...] @ w_ref[...] + b_ref[...]

    return pl.pallas_call(
        body,
        grid=(N // BN,),
        in_specs=[pl.BlockSpec((BN, H), lambda i: (i, 0)),
                  pl.BlockSpec((H, H), lambda i: (0, 0)),
                  pl.BlockSpec((1, H), lambda i: (0, 0))],
        out_specs=pl.BlockSpec((BN, H), lambda i: (i, 0)),
        out_shape=jax.ShapeDtypeStruct((N, H), jnp.float32),
    )(nf, W, b.reshape(1, H))


def _tc_logits(x, wa):
    def body(x_ref, wa_ref, a_ref, m_ref):
        a = x_ref[...] @ wa_ref[...]
        a_ref[...] = a
        m = jnp.max(a, axis=(0, 1), keepdims=True)

        @pl.when(pl.program_id(0) == 0)
        def _():
            m_ref[...] = m

        @pl.when(pl.program_id(0) != 0)
        def _():
            m_ref[...] = jnp.maximum(m_ref[...], m)

    return pl.pallas_call(
        body,
        grid=(N // BN,),
        in_specs=[pl.BlockSpec((BN, H), lambda i: (i, 0)),
                  pl.BlockSpec((H, 1), lambda i: (0, 0))],
        out_specs=[pl.BlockSpec((BN, 1), lambda i: (i, 0)),
                   pl.BlockSpec((1, 1), lambda i: (0, 0))],
        out_shape=[jax.ShapeDtypeStruct((N, 1), jnp.float32),
                   jax.ShapeDtypeStruct((1, 1), jnp.float32)],
    )(x, wa)


def _tc_table(x, a, m):
    def body(x_ref, a_ref, m_ref, t_ref):
        p = jnp.exp(a_ref[...] - m_ref[0, 0])
        t_ref[...] = jnp.concatenate(
            [x_ref[...] * p, p, jnp.zeros((BN, DT - H - 1), jnp.float32)],
            axis=1)

    return pl.pallas_call(
        body,
        grid=(N // BN,),
        in_specs=[pl.BlockSpec((BN, H), lambda i: (i, 0)),
                  pl.BlockSpec((BN, 1), lambda i: (i, 0)),
                  pl.BlockSpec((1, 1), lambda i: (0, 0))],
        out_specs=pl.BlockSpec((BN, DT), lambda i: (i, 0)),
        out_shape=jax.ShapeDtypeStruct((N, DT), jnp.float32),
    )(x, a, m)


def _tc_combine(S, x, wl):
    def body(s_ref, x_ref, wl_ref, o_ref):
        ss = s_ref[0] + s_ref[1]
        denom = ss[:, H:H + 1]
        agg = jnp.where(denom != 0.0, ss[:, :H] / denom, 0.0)
        h = x_ref[...] @ wl_ref[:H] + agg @ wl_ref[H:]
        o_ref[...] = jnp.maximum(h, 0.0)

    return pl.pallas_call(
        body,
        grid=(N // BN,),
        in_specs=[pl.BlockSpec((2, BN, DT), lambda i: (0, i, 0)),
                  pl.BlockSpec((BN, H), lambda i: (i, 0)),
                  pl.BlockSpec((2 * H, H), lambda i: (0, 0))],
        out_specs=pl.BlockSpec((BN, H), lambda i: (i, 0)),
        out_shape=jax.ShapeDtypeStruct((N, H), jnp.float32),
    )(S, x, wl)


def kernel(nf, edge_index, W_init, b_init, W_lin0, W_attn0, W_lin1, W_attn1):
    src = edge_index[0].astype(jnp.int32)
    dst = edge_index[1].astype(jnp.int32)
    pad = NW * R * CHUNK - E
    srcp = jnp.concatenate([src, jnp.zeros((pad,), jnp.int32)]).reshape(NW, R, CHUNK)
    dstp = jnp.concatenate([dst, jnp.full((pad,), TRASH, jnp.int32)]).reshape(NW, R, CHUNK)
    zblk = jnp.zeros((ZROWS, DT), jnp.float32)

    x = _tc_init(nf, W_init, b_init)
    for wl, wa in ((W_lin0, W_attn0), (W_lin1, W_attn1)):
        a, m = _tc_logits(x, wa[:H])
        T = _tc_table(x, a, m)
        S = _sc_edge_agg(T, srcp, dstp, zblk).reshape(2, NPAD, DT)
        x = _tc_combine(S, x, wl)
    return x
